# baseline (speedup 1.0000x reference)
import functools

import jax
import jax.numpy as jnp
from jax import lax
from jax.experimental import pallas as pl
from jax.experimental.pallas import tpu as pltpu
from jax.experimental.pallas import tpu_sc as plsc

_NC = 2
_NS = 16
_L = 16
_NW = _NC * _NS


def _make_body(num_edges, d_model):
    def body(edge_ref, ent_ref, rel_ref, out_ref,
             hidx, tidx, hbuf, tbuf, relv, outv, hsem, tsem):
        b_per_w = hidx.shape[0]
        n_groups = b_per_w // _L
        wid = lax.axis_index("s") * _NC + lax.axis_index("c")
        base = wid * b_per_w

        pltpu.sync_copy(edge_ref.at[pl.ds(base, b_per_w)], hidx)
        pltpu.sync_copy(edge_ref.at[pl.ds(num_edges + base, b_per_w)], tidx)
        pltpu.sync_copy(rel_ref, relv)

        lanes = lax.iota(jnp.int32, _L)
        lanes_d = lanes * d_model
        r_parts = [relv[pl.ds(c * _L, _L)] for c in range(d_model // _L)]
        rscal = [r_parts[d // _L][d % _L] for d in range(d_model)]

        def fetch(g):
            hv = hidx[pl.ds(g * _L, _L)] * d_model
            tv = tidx[pl.ds(g * _L, _L)] * d_model
            for j in range(_L):
                dst = pl.ds((g * _L + j) * d_model, d_model)
                hoff = pl.multiple_of(hv[j], 8)
                toff = pl.multiple_of(tv[j], 8)
                pltpu.async_copy(
                    ent_ref.at[pl.ds(hoff, d_model)], hbuf.at[dst], hsem)
                pltpu.async_copy(
                    ent_ref.at[pl.ds(toff, d_model)], tbuf.at[dst], tsem)

        def drain(g):
            dst = pl.ds(g * _L * d_model, _L * d_model)
            src = pl.ds(0, _L * d_model)
            pltpu.make_async_copy(ent_ref.at[src], hbuf.at[dst], hsem).wait()
            pltpu.make_async_copy(ent_ref.at[src], tbuf.at[dst], tsem).wait()

        def compute(g):
            acc = jnp.zeros((_L,), jnp.float32)
            for d in range(d_model):
                idx = lanes_d + (g * _L * d_model + d)
                hcol = plsc.load_gather(hbuf, [idx])
                tcol = plsc.load_gather(tbuf, [idx])
                acc = acc + hcol * tcol * rscal[d]
            sig = 1.0 / (1.0 + jnp.exp(-acc))
            outv[pl.ds(g * _L, _L)] = sig

        fetch(0)

        def step(g, carry):
            @pl.when(g + 1 < n_groups)
            def _():
                fetch(g + 1)
            drain(g)
            compute(g)
            return carry

        lax.fori_loop(0, n_groups, step, 0)
        pltpu.sync_copy(outv, out_ref.at[pl.ds(base, b_per_w)])

    return body


def kernel(edge_index, entity_emb, relation_emb):
    num_edges = edge_index.shape[1]
    d_model = entity_emb.shape[1]
    b_per_w = num_edges // _NW
    mesh = plsc.VectorSubcoreMesh(core_axis_name="c", subcore_axis_name="s")
    k = functools.partial(
        pl.kernel,
        mesh=mesh,
        out_type=jax.ShapeDtypeStruct((num_edges,), jnp.float32),
        compiler_params=pltpu.CompilerParams(
            needs_layout_passes=False, use_tc_tiling_on_sc=False),
        scratch_types=[
            pltpu.VMEM((b_per_w,), jnp.int32),
            pltpu.VMEM((b_per_w,), jnp.int32),
            pltpu.VMEM((b_per_w * d_model,), jnp.float32),
            pltpu.VMEM((b_per_w * d_model,), jnp.float32),
            pltpu.VMEM((d_model,), jnp.float32),
            pltpu.VMEM((b_per_w,), jnp.float32),
            pltpu.SemaphoreType.DMA,
            pltpu.SemaphoreType.DMA,
        ],
    )(_make_body(num_edges, d_model))
    return k(edge_index.reshape(-1), entity_emb.reshape(-1),
             relation_emb.reshape(-1))
